# baseline (speedup 1.0000x reference)
import functools

import jax
import jax.numpy as jnp
from jax import lax
from jax.experimental import pallas as pl
from jax.experimental.pallas import tpu as pltpu
from jax.experimental.pallas import tpu_sc as plsc

_HID = 512
_CBK = 1024
_NPTS = 2048
_B = 8
_RES = 12
_NVOX = _RES * _RES * _RES
_NVOXP = 1792
_NCELL = 216
_UPS = [4, 8, 8, 8]
_SCALES = [1, 4, 32, 256, 2048]
_F32 = jnp.float32


def _pick_bm(m):
    if m <= 512:
        return m
    for c in (256, 512, 432, 216, 128, 64, 32, 16, 8):
        if m % c == 0:
            return c
    return m


def _mlp_chain(x, specs, extra=None, mode=None):
    m, k0 = x.shape
    bm = _pick_bm(m)
    grid = (m // bm,)
    nlayers = len(specs)
    relus = tuple(bool(s[2]) for s in specs)
    kout = specs[-1][0].shape[1]

    def body(*refs):
        x_ref = refs[0]
        wrefs = refs[1:1 + 2 * nlayers]
        pos = 1 + 2 * nlayers
        ex_ref = refs[pos] if extra is not None else None
        o_ref = refs[-1]
        h = x_ref[...]
        for j in range(nlayers):
            w = wrefs[2 * j][...]
            bb = wrefs[2 * j + 1][...]
            h = jnp.dot(h, w, preferred_element_type=_F32) + bb
            if relus[j]:
                h = jnp.maximum(h, 0.0)
        if mode == 'add':
            h = h + ex_ref[...]
        elif mode == 'rsub':
            h = ex_ref[...] - h
        o_ref[...] = h

    in_specs = [pl.BlockSpec((bm, k0), lambda i: (i, 0))]
    args = [x]
    for (w, b, _r) in specs:
        in_specs.append(pl.BlockSpec(w.shape, lambda i: (0, 0)))
        args.append(w)
        b2 = b.reshape(1, -1)
        in_specs.append(pl.BlockSpec(b2.shape, lambda i: (0, 0)))
        args.append(b2)
    if extra is not None:
        in_specs.append(pl.BlockSpec((bm, kout), lambda i: (i, 0)))
        args.append(extra)
    return pl.pallas_call(
        body,
        grid=grid,
        in_specs=in_specs,
        out_specs=pl.BlockSpec((bm, kout), lambda i: (i, 0)),
        out_shape=jax.ShapeDtypeStruct((m, kout), _F32),
    )(*args)


def _row_index(f3t, perm):
    b = f3t.shape[0]
    nsel = None if perm is None else perm.shape[0]

    def body(*refs):
        if nsel is None:
            x_ref, o_ref = refs
        else:
            x_ref, p_ref, o_ref = refs
        x = x_ref[0]
        mn = jnp.min(x, axis=1, keepdims=True)
        mx = jnp.max(x, axis=1, keepdims=True)
        xn = (x - mn) / (mx - mn) * (_RES - 1)
        xi = jnp.clip(xn.astype(jnp.int32), 0, _RES - 1)
        d = xi[0:1]
        h = xi[1:2]
        w = xi[2:3]
        cell = (d // 2) * 36 + (h // 2) * 6 + (w // 2)
        off = (d % 2) * 4 + (h % 2) * 2 + (w % 2)
        rows = cell * 8 + off
        if nsel is not None:
            keep = cell == p_ref[0]
            for i in range(1, nsel):
                keep = keep | (cell == p_ref[i])
            rows = jnp.where(keep, rows, _NVOXP - 1)
        o_ref[...] = rows[None]

    in_specs = [pl.BlockSpec((1, 3, _NPTS), lambda i: (i, 0, 0))]
    args = [f3t]
    if nsel is not None:
        in_specs.append(pl.BlockSpec(memory_space=pltpu.SMEM))
        args.append(perm.astype(jnp.int32))
    return pl.pallas_call(
        body,
        grid=(b,),
        in_specs=in_specs,
        out_specs=pl.BlockSpec((1, 1, _NPTS), lambda i: (i, 0, 0)),
        out_shape=jax.ShapeDtypeStruct((b, 1, _NPTS), jnp.int32),
    )(*args)


def _make_scatter(skip):
    mesh = plsc.VectorSubcoreMesh(core_axis_name="c", subcore_axis_name="s")
    cg = _HID // 16
    chunk = 128

    @functools.partial(
        pl.kernel,
        out_type=jax.ShapeDtypeStruct((_B, _NVOXP, _HID), _F32),
        mesh=mesh,
        scratch_types=[
            pltpu.VMEM((_NVOXP, cg), _F32),
            pltpu.VMEM((2, chunk, cg), _F32),
            pltpu.VMEM((_NPTS,), jnp.int32),
            pltpu.SemaphoreType.DMA((2,)),
        ],
        compiler_params=pltpu.CompilerParams(use_tc_tiling_on_sc=False,
                                             needs_layout_passes=False),
    )
    def scatter(f_hbm, rows_hbm, out_hbm, acc, pts, idx, dsem):
        ci = lax.axis_index("c")
        si = lax.axis_index("s")
        c0 = si * cg
        lanes = lax.broadcasted_iota(jnp.int32, (16,), 0)
        zv = jnp.zeros((16,), _F32)

        def zrow(r, carry):
            for u in range(8):
                acc[r * 8 + u, pl.ds(0, 16)] = zv
                acc[r * 8 + u, pl.ds(16, 16)] = zv
            return carry

        def process(buf, off):
            def group_body(g, rows16):
                for l in range(16):
                    rv = jnp.full((16,), rows16[l], jnp.int32)
                    n = g * 16 + l
                    plsc.addupdate_scatter(
                        acc, [rv, lanes], pts[buf, n, pl.ds(0, 16)])
                    plsc.addupdate_scatter(
                        acc, [rv, lanes + 16], pts[buf, n, pl.ds(16, 16)])

            def group(g, carry):
                rows16 = idx[pl.ds(off + g * 16, 16)]
                if skip:
                    @pl.when(jnp.min(rows16) < _NVOXP - 1)
                    def _():
                        group_body(g, rows16)
                else:
                    group_body(g, rows16)
                return carry

            lax.fori_loop(0, chunk // 16, group, 0)

        def round_body(t, carry):
            b = ci * (_B // 2) + t
            lax.fori_loop(0, _NVOXP // 8, zrow, 0)
            pltpu.sync_copy(rows_hbm.at[b], idx)
            pltpu.async_copy(f_hbm.at[b, pl.ds(0, chunk), pl.ds(c0, cg)],
                             pts.at[0], dsem.at[0])

            def pair(h, carry2):
                o0 = pl.multiple_of(h * 2 * chunk, 2 * chunk)
                pltpu.make_async_copy(
                    f_hbm.at[b, pl.ds(o0, chunk), pl.ds(c0, cg)],
                    pts.at[0], dsem.at[0]).wait()
                pltpu.async_copy(
                    f_hbm.at[b, pl.ds(o0 + chunk, chunk), pl.ds(c0, cg)],
                    pts.at[1], dsem.at[1])
                process(0, o0)
                pltpu.make_async_copy(
                    f_hbm.at[b, pl.ds(o0 + chunk, chunk), pl.ds(c0, cg)],
                    pts.at[1], dsem.at[1]).wait()

                @pl.when(h < _NPTS // (2 * chunk) - 1)
                def _():
                    pltpu.async_copy(
                        f_hbm.at[b, pl.ds(o0 + 2 * chunk, chunk),
                                 pl.ds(c0, cg)],
                        pts.at[0], dsem.at[0])

                process(1, o0 + chunk)
                return carry2

            lax.fori_loop(0, _NPTS // (2 * chunk), pair, 0)
            pltpu.sync_copy(acc, out_hbm.at[b, :, pl.ds(c0, cg)])
            return carry

        lax.fori_loop(0, _B // 2, round_body, 0)

    return scatter


_scatter_cache = {}


def _scatter_call(f3d, rows, skip):
    if skip not in _scatter_cache:
        _scatter_cache[skip] = _make_scatter(skip)
    return _scatter_cache[skip](f3d, rows)


def _conv_vq(patches, sel, wc, bc, emb, embt):
    b = patches.shape[0]
    n = _NCELL if sel is None else sel.shape[0]
    have_sel = sel is not None

    def body(*refs):
        if have_sel:
            (p_ref, s_ref, wc_ref, bc_ref, emb_ref, embt_ref,
             q_ref, l_ref) = refs
        else:
            p_ref, wc_ref, bc_ref, emb_ref, embt_ref, q_ref, l_ref = refs
        xp = p_ref[0]
        if have_sel:
            xp = jnp.dot(s_ref[...], xp, preferred_element_type=_F32)
        y = jnp.dot(xp, wc_ref[...], preferred_element_type=_F32) + bc_ref[...]
        et = embt_ref[...]
        esq = jnp.sum(et * et, axis=0, keepdims=True)
        d = (jnp.sum(y * y, axis=1, keepdims=True) + esq
             - 2.0 * jnp.dot(y, et, preferred_element_type=_F32))
        mn = jnp.min(d, axis=1, keepdims=True)
        io = lax.broadcasted_iota(jnp.int32, d.shape, 1)
        idx = jnp.min(jnp.where(d == mn, io, jnp.int32(2 ** 30)),
                      axis=1, keepdims=True)
        oh = (io == idx).astype(_F32)
        q = jnp.dot(oh, emb_ref[...], preferred_element_type=_F32)
        q_ref[0] = q
        l_ref[...] = jnp.full((1, 1, 128), jnp.sum((q - y) ** 2), _F32)

    in_specs = [pl.BlockSpec((1, _NCELL, 8 * _HID), lambda i: (i, 0, 0))]
    args = [patches]
    if have_sel:
        in_specs.append(pl.BlockSpec(sel.shape, lambda i: (0, 0)))
        args.append(sel)
    bc2 = bc.reshape(1, -1)
    for a in (wc, bc2, emb, embt):
        in_specs.append(pl.BlockSpec(a.shape, lambda i: (0, 0)))
        args.append(a)
    return pl.pallas_call(
        body,
        grid=(b,),
        in_specs=in_specs,
        out_specs=[
            pl.BlockSpec((1, n, _HID), lambda i: (i, 0, 0)),
            pl.BlockSpec((1, 1, 128), lambda i: (i, 0, 0)),
        ],
        out_shape=[
            jax.ShapeDtypeStruct((b, n, _HID), _F32),
            jax.ShapeDtypeStruct((b, 1, 128), _F32),
        ],
    )(*args)


def _chamfer_sums(xp, rt):
    b = xp.shape[0]
    nr = rt.shape[2]

    def body(x_ref, r_ref, s1_ref, s0_ref):
        xb = x_ref[0]
        rb = r_ref[0]
        xx = jnp.sum(xb * xb, axis=1, keepdims=True)
        yy = jnp.sum(rb * rb, axis=0, keepdims=True)
        pmat = xx + yy - 2.0 * jnp.dot(xb, rb, preferred_element_type=_F32)
        s1 = jnp.sum(jnp.min(pmat, axis=1))
        s0 = jnp.sum(jnp.min(pmat, axis=0))
        s1_ref[...] = jnp.full((1, 1, 128), s1, _F32)
        s0_ref[...] = jnp.full((1, 1, 128), s0, _F32)

    return pl.pallas_call(
        body,
        grid=(b,),
        in_specs=[
            pl.BlockSpec((1, _NPTS, 128), lambda i: (i, 0, 0)),
            pl.BlockSpec((1, 128, nr), lambda i: (i, 0, 0)),
        ],
        out_specs=[
            pl.BlockSpec((1, 1, 128), lambda i: (i, 0, 0)),
            pl.BlockSpec((1, 1, 128), lambda i: (i, 0, 0)),
        ],
        out_shape=[
            jax.ShapeDtypeStruct((b, 1, 128), _F32),
            jax.ShapeDtypeStruct((b, 1, 128), _F32),
        ],
    )(xp, rt)


def _mlp2_spec(blk):
    return [(blk[0][0], blk[0][1], True), (blk[1][0], blk[1][1], False)]


def _ups_plan(n, target):
    steps = []
    cur = n
    while cur < target:
        factor = min(_UPS, key=lambda f: abs(f - target // cur))
        steps.append((_UPS.index(factor), factor))
        cur *= factor
    return steps


def _ups_rows(ups_params, q3d, target, need):
    b, n = q3d.shape[0], q3d.shape[1]
    if n >= target:
        return q3d[:, :need]
    steps = _ups_plan(n, target)
    in_rows = [0] * len(steps)
    out_need = [0] * len(steps)
    r = need
    for i in reversed(range(len(steps))):
        out_need[i] = r
        r = -(-r // steps[i][1])
        in_rows[i] = r
    cur = q3d[:, :in_rows[0]]
    for i, (mi, fct) in enumerate(steps):
        rows2 = cur.reshape(b * cur.shape[1], _HID)
        o = _mlp_chain(rows2, _mlp2_spec(ups_params[mi]))
        cur = o.reshape(b, cur.shape[1] * fct, _HID)[:, :out_need[i]]
    return cur


def kernel(x, params):
    p = params
    b, npts = x.shape[0], x.shape[1]

    x2 = x.reshape(-1, 3)
    xpad = jnp.pad(x2, ((0, 0), (0, 125)))
    w1 = jnp.pad(p['enc'][0][0], ((0, 125), (0, 0)))
    enc_specs = [(w1, p['enc'][0][1], True),
                 (p['enc'][1][0], p['enc'][1][1], True),
                 (p['enc'][2][0], p['enc'][2][1], False)]
    f2 = _mlp_chain(xpad, enc_specs)

    wc = p['conv_w'].transpose(2, 3, 4, 1, 0).reshape(8 * _HID, _HID)
    bc = p['conv_b']
    emb = p['emb']
    embt = emb.T

    sels = []
    perms = []
    for k in range(5):
        npnt = _SCALES[k]
        if _NCELL > npnt:
            perm = jax.random.permutation(
                jax.random.key(1000 + npnt), _NCELL)[:npnt]
            sels.append((perm[:, None] == jnp.arange(_NCELL)[None, :])
                        .astype(_F32))
            perms.append(perm)
        else:
            sels.append(None)
            perms.append(None)

    qs = []
    loss_parts = []
    for k in range(5):
        f3d = f2.reshape(b, npts, _HID)
        f3t = jnp.transpose(f3d[:, :, :3], (0, 2, 1))
        rows = _row_index(f3t, perms[k]).reshape(b, npts)
        vox = _scatter_call(f3d, rows, k < 2)[:, :_NVOX]
        patches = vox.reshape(b, _NCELL, 8 * _HID)
        q, lsum = _conv_vq(patches, sels[k], wc, bc, emb, embt)
        nk = q.shape[1]
        loss_parts.append(1.25 * jnp.sum(lsum[:, 0, 0]) / (b * nk * _HID))
        qs.append(q)
        if k < 4:
            z = _ups_rows(p['ups'], q, npts, npts)
            f2 = _mlp_chain(z.reshape(-1, _HID), _mlp2_spec(p['phi'][k]),
                            extra=f2, mode='rsub')

    vq_loss = loss_parts[0]
    for l in loss_parts[1:]:
        vq_loss = vq_loss + l

    fh = None
    for k in range(5):
        tgt = _SCALES[k]
        nk = qs[k].shape[1]
        m = min(tgt, nk)
        phi_blk = p['phi'][k - 1]
        if k == 0:
            fh = _mlp_chain(qs[0].reshape(-1, _HID),
                            _mlp2_spec(phi_blk)).reshape(b, 1, _HID)
        else:
            z = _ups_rows(p['ups'], fh, tgt, m)
            fh = _mlp_chain(qs[k][:, :m].reshape(-1, _HID),
                            _mlp2_spec(phi_blk),
                            extra=z.reshape(-1, _HID),
                            mode='add').reshape(b, m, _HID)

    dec_specs = []
    for blk in p['dec_layers']:
        dec_specs += _mlp2_spec(blk)
    wf = jnp.pad(p['dec_final'][1][0], ((0, 0), (0, 125)))
    bf = jnp.pad(p['dec_final'][1][1], ((0, 125),))
    dec_specs += [(p['dec_final'][0][0], p['dec_final'][0][1], True),
                  (wf, bf, False)]
    nrec = fh.shape[1]
    rec_pad = _mlp_chain(fh.reshape(-1, _HID), dec_specs)
    rec_pad = rec_pad.reshape(b, nrec, 128)
    recon = rec_pad[:, :, :3]

    xp = jnp.pad(x, ((0, 0), (0, 0), (0, 125)))
    rt = jnp.transpose(rec_pad, (0, 2, 1))
    s1, s0 = _chamfer_sums(xp, rt)
    ch = (jnp.sum(s1[:, 0, 0]) / (b * npts)
          + jnp.sum(s0[:, 0, 0]) / (b * nrec))

    return recon, ch + vq_loss, ch, vq_loss


# speedup vs baseline: 1.0904x; 1.0904x over previous
import functools

import jax
import jax.numpy as jnp
from jax import lax
from jax.experimental import pallas as pl
from jax.experimental.pallas import tpu as pltpu
from jax.experimental.pallas import tpu_sc as plsc

_HID = 512
_CBK = 1024
_NPTS = 2048
_B = 8
_RES = 12
_NVOX = _RES * _RES * _RES
_NVOXP = 1792
_NCELL = 216
_UPS = [4, 8, 8, 8]
_SCALES = [1, 4, 32, 256, 2048]
_F32 = jnp.float32


def _pick_bm(m):
    if m <= 512:
        return m
    for c in (1024, 512, 432, 216, 128, 64, 32, 16, 8):
        if m % c == 0:
            return c
    return m


def _mlp_chain(x, specs, extra=None, mode=None):
    m, k0 = x.shape
    bm = _pick_bm(m)
    grid = (m // bm,)
    nlayers = len(specs)
    relus = tuple(bool(s[2]) for s in specs)
    kout = specs[-1][0].shape[1]

    def body(*refs):
        x_ref = refs[0]
        wrefs = refs[1:1 + 2 * nlayers]
        pos = 1 + 2 * nlayers
        ex_ref = refs[pos] if extra is not None else None
        o_ref = refs[-1]
        h = x_ref[...]
        for j in range(nlayers):
            w = wrefs[2 * j][...]
            bb = wrefs[2 * j + 1][...]
            h = jnp.dot(h, w, preferred_element_type=_F32) + bb
            if relus[j]:
                h = jnp.maximum(h, 0.0)
        if mode == 'add':
            h = h + ex_ref[...]
        elif mode == 'rsub':
            h = ex_ref[...] - h
        o_ref[...] = h

    in_specs = [pl.BlockSpec((bm, k0), lambda i: (i, 0))]
    args = [x]
    for (w, b, _r) in specs:
        in_specs.append(pl.BlockSpec(w.shape, lambda i: (0, 0)))
        args.append(w)
        b2 = b.reshape(1, -1)
        in_specs.append(pl.BlockSpec(b2.shape, lambda i: (0, 0)))
        args.append(b2)
    if extra is not None:
        in_specs.append(pl.BlockSpec((bm, kout), lambda i: (i, 0)))
        args.append(extra)
    return pl.pallas_call(
        body,
        grid=grid,
        in_specs=in_specs,
        out_specs=pl.BlockSpec((bm, kout), lambda i: (i, 0)),
        out_shape=jax.ShapeDtypeStruct((m, kout), _F32),
    )(*args)


def _row_index(f3t, perm):
    b = f3t.shape[0]
    nsel = None if perm is None else perm.shape[0]

    def body(*refs):
        if nsel is None:
            x_ref, o_ref = refs
        else:
            x_ref, p_ref, o_ref = refs
        x = x_ref[0]
        mn = jnp.min(x, axis=1, keepdims=True)
        mx = jnp.max(x, axis=1, keepdims=True)
        xn = (x - mn) / (mx - mn) * (_RES - 1)
        xi = jnp.clip(xn.astype(jnp.int32), 0, _RES - 1)
        d = xi[0:1]
        h = xi[1:2]
        w = xi[2:3]
        cell = (d // 2) * 36 + (h // 2) * 6 + (w // 2)
        off = (d % 2) * 4 + (h % 2) * 2 + (w % 2)
        rows = cell * 8 + off
        if nsel is not None:
            keep = cell == p_ref[0]
            for i in range(1, nsel):
                keep = keep | (cell == p_ref[i])
            rows = jnp.where(keep, rows, _NVOXP - 1)
        o_ref[...] = rows[None]

    in_specs = [pl.BlockSpec((1, 3, _NPTS), lambda i: (i, 0, 0))]
    args = [f3t]
    if nsel is not None:
        in_specs.append(pl.BlockSpec(memory_space=pltpu.SMEM))
        args.append(perm.astype(jnp.int32))
    return pl.pallas_call(
        body,
        grid=(b,),
        in_specs=in_specs,
        out_specs=pl.BlockSpec((1, 1, _NPTS), lambda i: (i, 0, 0)),
        out_shape=jax.ShapeDtypeStruct((b, 1, _NPTS), jnp.int32),
    )(*args)


def _make_scatter(skip):
    mesh = plsc.VectorSubcoreMesh(core_axis_name="c", subcore_axis_name="s")
    cg = _HID // 16
    chunk = 128

    @functools.partial(
        pl.kernel,
        out_type=jax.ShapeDtypeStruct((_B, _NVOXP, _HID), _F32),
        mesh=mesh,
        scratch_types=[
            pltpu.VMEM((_NVOXP, cg), _F32),
            pltpu.VMEM((2, chunk, cg), _F32),
            pltpu.VMEM((_NPTS,), jnp.int32),
            pltpu.SemaphoreType.DMA((2,)),
        ],
        compiler_params=pltpu.CompilerParams(use_tc_tiling_on_sc=False,
                                             needs_layout_passes=False),
    )
    def scatter(f_hbm, rows_hbm, out_hbm, acc, pts, idx, dsem):
        ci = lax.axis_index("c")
        si = lax.axis_index("s")
        c0 = si * cg
        lanes = lax.broadcasted_iota(jnp.int32, (16,), 0)
        zv = jnp.zeros((16,), _F32)

        def zrow(r, carry):
            for u in range(8):
                acc[r * 8 + u, pl.ds(0, 16)] = zv
                acc[r * 8 + u, pl.ds(16, 16)] = zv
            return carry

        def process(buf, off):
            def group_body(g, rows16):
                for l in range(16):
                    rv = jnp.full((16,), rows16[l], jnp.int32)
                    n = g * 16 + l
                    plsc.addupdate_scatter(
                        acc, [rv, lanes], pts[buf, n, pl.ds(0, 16)])
                    plsc.addupdate_scatter(
                        acc, [rv, lanes + 16], pts[buf, n, pl.ds(16, 16)])

            def group(g, carry):
                rows16 = idx[pl.ds(off + g * 16, 16)]
                if skip:
                    @pl.when(jnp.min(rows16) < _NVOXP - 1)
                    def _():
                        group_body(g, rows16)
                else:
                    group_body(g, rows16)
                return carry

            lax.fori_loop(0, chunk // 16, group, 0)

        def round_body(t, carry):
            b = ci * (_B // 2) + t
            lax.fori_loop(0, _NVOXP // 8, zrow, 0)
            pltpu.sync_copy(rows_hbm.at[b], idx)
            pltpu.async_copy(f_hbm.at[b, pl.ds(0, chunk), pl.ds(c0, cg)],
                             pts.at[0], dsem.at[0])

            def pair(h, carry2):
                o0 = pl.multiple_of(h * 2 * chunk, 2 * chunk)
                pltpu.make_async_copy(
                    f_hbm.at[b, pl.ds(o0, chunk), pl.ds(c0, cg)],
                    pts.at[0], dsem.at[0]).wait()
                pltpu.async_copy(
                    f_hbm.at[b, pl.ds(o0 + chunk, chunk), pl.ds(c0, cg)],
                    pts.at[1], dsem.at[1])
                process(0, o0)
                pltpu.make_async_copy(
                    f_hbm.at[b, pl.ds(o0 + chunk, chunk), pl.ds(c0, cg)],
                    pts.at[1], dsem.at[1]).wait()

                @pl.when(h < _NPTS // (2 * chunk) - 1)
                def _():
                    pltpu.async_copy(
                        f_hbm.at[b, pl.ds(o0 + 2 * chunk, chunk),
                                 pl.ds(c0, cg)],
                        pts.at[0], dsem.at[0])

                process(1, o0 + chunk)
                return carry2

            lax.fori_loop(0, _NPTS // (2 * chunk), pair, 0)
            pltpu.sync_copy(acc, out_hbm.at[b, :, pl.ds(c0, cg)])
            return carry

        lax.fori_loop(0, _B // 2, round_body, 0)

    return scatter


_scatter_cache = {}


def _scatter_call(f3d, rows, skip):
    if skip not in _scatter_cache:
        _scatter_cache[skip] = _make_scatter(skip)
    return _scatter_cache[skip](f3d, rows)


def _conv_vq(patches, sel, wc, bc, emb, embt):
    b = patches.shape[0]
    n = _NCELL if sel is None else sel.shape[0]
    have_sel = sel is not None

    def body(*refs):
        if have_sel:
            (p_ref, s_ref, wc_ref, bc_ref, emb_ref, embt_ref,
             q_ref, l_ref) = refs
        else:
            p_ref, wc_ref, bc_ref, emb_ref, embt_ref, q_ref, l_ref = refs
        xp = p_ref[0]
        if have_sel:
            xp = jnp.dot(s_ref[...], xp, preferred_element_type=_F32)
        y = jnp.dot(xp, wc_ref[...], preferred_element_type=_F32) + bc_ref[...]
        et = embt_ref[...]
        esq = jnp.sum(et * et, axis=0, keepdims=True)
        d = (jnp.sum(y * y, axis=1, keepdims=True) + esq
             - 2.0 * jnp.dot(y, et, preferred_element_type=_F32))
        mn = jnp.min(d, axis=1, keepdims=True)
        io = lax.broadcasted_iota(jnp.int32, d.shape, 1)
        idx = jnp.min(jnp.where(d == mn, io, jnp.int32(2 ** 30)),
                      axis=1, keepdims=True)
        oh = (io == idx).astype(_F32)
        q = jnp.dot(oh, emb_ref[...], preferred_element_type=_F32)
        q_ref[0] = q
        l_ref[...] = jnp.full((1, 1, 128), jnp.sum((q - y) ** 2), _F32)

    in_specs = [pl.BlockSpec((1, _NCELL, 8 * _HID), lambda i: (i, 0, 0))]
    args = [patches]
    if have_sel:
        in_specs.append(pl.BlockSpec(sel.shape, lambda i: (0, 0)))
        args.append(sel)
    bc2 = bc.reshape(1, -1)
    for a in (wc, bc2, emb, embt):
        in_specs.append(pl.BlockSpec(a.shape, lambda i: (0, 0)))
        args.append(a)
    return pl.pallas_call(
        body,
        grid=(b,),
        in_specs=in_specs,
        out_specs=[
            pl.BlockSpec((1, n, _HID), lambda i: (i, 0, 0)),
            pl.BlockSpec((1, 1, 128), lambda i: (i, 0, 0)),
        ],
        out_shape=[
            jax.ShapeDtypeStruct((b, n, _HID), _F32),
            jax.ShapeDtypeStruct((b, 1, 128), _F32),
        ],
    )(*args)


def _chamfer_sums(xp, rt):
    b = xp.shape[0]
    nr = rt.shape[2]

    def body(x_ref, r_ref, s1_ref, s0_ref):
        xb = x_ref[0]
        rb = r_ref[0]
        xx = jnp.sum(xb * xb, axis=1, keepdims=True)
        yy = jnp.sum(rb * rb, axis=0, keepdims=True)
        pmat = xx + yy - 2.0 * jnp.dot(xb, rb, preferred_element_type=_F32)
        s1 = jnp.sum(jnp.min(pmat, axis=1))
        s0 = jnp.sum(jnp.min(pmat, axis=0))
        s1_ref[...] = jnp.full((1, 1, 128), s1, _F32)
        s0_ref[...] = jnp.full((1, 1, 128), s0, _F32)

    return pl.pallas_call(
        body,
        grid=(b,),
        in_specs=[
            pl.BlockSpec((1, _NPTS, 128), lambda i: (i, 0, 0)),
            pl.BlockSpec((1, 128, nr), lambda i: (i, 0, 0)),
        ],
        out_specs=[
            pl.BlockSpec((1, 1, 128), lambda i: (i, 0, 0)),
            pl.BlockSpec((1, 1, 128), lambda i: (i, 0, 0)),
        ],
        out_shape=[
            jax.ShapeDtypeStruct((b, 1, 128), _F32),
            jax.ShapeDtypeStruct((b, 1, 128), _F32),
        ],
    )(xp, rt)


def _mlp2_spec(blk):
    return [(blk[0][0], blk[0][1], True), (blk[1][0], blk[1][1], False)]


def _ups_plan(n, target):
    steps = []
    cur = n
    while cur < target:
        factor = min(_UPS, key=lambda f: abs(f - target // cur))
        steps.append((_UPS.index(factor), factor))
        cur *= factor
    return steps


def _ups_rows(ups_params, q3d, target, need):
    b, n = q3d.shape[0], q3d.shape[1]
    if n >= target:
        return q3d[:, :need]
    steps = _ups_plan(n, target)
    in_rows = [0] * len(steps)
    out_need = [0] * len(steps)
    r = need
    for i in reversed(range(len(steps))):
        out_need[i] = r
        r = -(-r // steps[i][1])
        in_rows[i] = r
    cur = q3d[:, :in_rows[0]]
    for i, (mi, fct) in enumerate(steps):
        rows2 = cur.reshape(b * cur.shape[1], _HID)
        o = _mlp_chain(rows2, _mlp2_spec(ups_params[mi]))
        cur = o.reshape(b, cur.shape[1] * fct, _HID)[:, :out_need[i]]
    return cur


def kernel(x, params):
    p = params
    b, npts = x.shape[0], x.shape[1]

    x2 = x.reshape(-1, 3)
    xpad = jnp.pad(x2, ((0, 0), (0, 125)))
    w1 = jnp.pad(p['enc'][0][0], ((0, 125), (0, 0)))
    enc_specs = [(w1, p['enc'][0][1], True),
                 (p['enc'][1][0], p['enc'][1][1], True),
                 (p['enc'][2][0], p['enc'][2][1], False)]
    f2 = _mlp_chain(xpad, enc_specs)

    wc = p['conv_w'].transpose(2, 3, 4, 1, 0).reshape(8 * _HID, _HID)
    bc = p['conv_b']
    emb = p['emb']
    embt = emb.T

    sels = []
    perms = []
    for k in range(5):
        npnt = _SCALES[k]
        if _NCELL > npnt:
            perm = jax.random.permutation(
                jax.random.key(1000 + npnt), _NCELL)[:npnt]
            sels.append((perm[:, None] == jnp.arange(_NCELL)[None, :])
                        .astype(_F32))
            perms.append(perm)
        else:
            sels.append(None)
            perms.append(None)

    qs = []
    loss_parts = []
    for k in range(5):
        f3d = f2.reshape(b, npts, _HID)
        f3t = jnp.transpose(f3d[:, :, :3], (0, 2, 1))
        rows = _row_index(f3t, perms[k]).reshape(b, npts)
        vox = _scatter_call(f3d, rows, k < 2)[:, :_NVOX]
        patches = vox.reshape(b, _NCELL, 8 * _HID)
        q, lsum = _conv_vq(patches, sels[k], wc, bc, emb, embt)
        nk = q.shape[1]
        loss_parts.append(1.25 * jnp.sum(lsum[:, 0, 0]) / (b * nk * _HID))
        qs.append(q)
        if k < 4:
            z = _ups_rows(p['ups'], q, npts, npts)
            f2 = _mlp_chain(z.reshape(-1, _HID), _mlp2_spec(p['phi'][k]),
                            extra=f2, mode='rsub')

    vq_loss = loss_parts[0]
    for l in loss_parts[1:]:
        vq_loss = vq_loss + l

    fh = None
    for k in range(5):
        tgt = _SCALES[k]
        nk = qs[k].shape[1]
        m = min(tgt, nk)
        phi_blk = p['phi'][k - 1]
        if k == 0:
            fh = _mlp_chain(qs[0].reshape(-1, _HID),
                            _mlp2_spec(phi_blk)).reshape(b, 1, _HID)
        else:
            z = _ups_rows(p['ups'], fh, tgt, m)
            fh = _mlp_chain(qs[k][:, :m].reshape(-1, _HID),
                            _mlp2_spec(phi_blk),
                            extra=z.reshape(-1, _HID),
                            mode='add').reshape(b, m, _HID)

    dec_specs = []
    for blk in p['dec_layers']:
        dec_specs += _mlp2_spec(blk)
    wf = jnp.pad(p['dec_final'][1][0], ((0, 0), (0, 125)))
    bf = jnp.pad(p['dec_final'][1][1], ((0, 125),))
    dec_specs += [(p['dec_final'][0][0], p['dec_final'][0][1], True),
                  (wf, bf, False)]
    nrec = fh.shape[1]
    rec_pad = _mlp_chain(fh.reshape(-1, _HID), dec_specs)
    rec_pad = rec_pad.reshape(b, nrec, 128)
    recon = rec_pad[:, :, :3]

    xp = jnp.pad(x, ((0, 0), (0, 0), (0, 125)))
    rt = jnp.transpose(rec_pad, (0, 2, 1))
    s1, s0 = _chamfer_sums(xp, rt)
    ch = (jnp.sum(s1[:, 0, 0]) / (b * npts)
          + jnp.sum(s0[:, 0, 0]) / (b * nrec))

    return recon, ch + vq_loss, ch, vq_loss
